# Initial kernel scaffold; baseline (speedup 1.0000x reference)
import jax
import jax.numpy as jnp
from jax.experimental import pallas as pl


def kernel(x, edge_index, W_in, b_in, W1, b1, ln1_s, ln1_b, W2, b2, ln2_s, ln2_b, Watt1, batt1, Watt2, batt2, Wc1, bc1, Wc2, bc2):
    raise NotImplementedError("write your pallas kernel here")


# speedup vs baseline: 11.6057x; 11.6057x over previous
import functools

import jax
import jax.numpy as jnp
from jax import lax
from jax.experimental import pallas as pl
from jax.experimental.pallas import tpu as pltpu
from jax.experimental.pallas import tpu_sc as plsc

_N = 10000
_NP = 10240
_E = 320000
_HF = 128
_NC, _NS = 2, 16
_CH = 80
_NCHUNK = _E // _CH
_CPS = _NCHUNK // _NS
_GRP = 50
_NGRP = _CPS // _GRP
_DPW = _NCHUNK // (_NC * _NS)
_RPS = _NP // _NS
_ZR = 64
_NZ = _RPS // _ZR


def _sc_mesh():
    return plsc.VectorSubcoreMesh(
        core_axis_name="c", subcore_axis_name="s",
        num_cores=_NC, num_subcores=_NS)


def _sc_degree(dsts_d, z128, ones128):
    @functools.partial(
        pl.kernel,
        out_type=jax.ShapeDtypeStruct((2 * _NP, _HF), jnp.float32),
        mesh=_sc_mesh(),
        scratch_types=[
            pltpu.VMEM((_DPW, _CH), jnp.int32),
            pltpu.VMEM((_CH, _HF), jnp.float32),
            pltpu.VMEM((_ZR, _HF), jnp.float32),
            pltpu.VMEM_SHARED((_NP, _HF), jnp.float32),
        ],
    )
    def k(dsts_hbm, z_hbm, ones_hbm, out_hbm, didx, ones_v, zbuf, acc):
        c = lax.axis_index("c")
        s = lax.axis_index("s")
        wid = c * _NS + s
        pltpu.sync_copy(dsts_hbm.at[wid], didx)
        pltpu.sync_copy(ones_hbm, ones_v)
        pltpu.sync_copy(z_hbm, zbuf)
        for t in range(_NZ):
            pltpu.sync_copy(zbuf, acc.at[pl.ds(s * _RPS + t * _ZR, _ZR)])
        plsc.subcore_barrier()

        def body(j, carry):
            pltpu.sync_copy(ones_v, acc.at[didx.at[j]], add=True)
            return carry

        lax.fori_loop(0, _DPW, body, 0)
        plsc.subcore_barrier()
        for t in range(_NZ):
            pltpu.sync_copy(acc.at[pl.ds(s * _RPS + t * _ZR, _ZR)], zbuf)
            pltpu.sync_copy(
                zbuf, out_hbm.at[pl.ds(c * _NP + s * _RPS + t * _ZR, _ZR)])

    return k(dsts_d, z128, ones128)


def _sc_scatter(g2, srcs2, dsts_s, z128):
    @functools.partial(
        pl.kernel,
        out_type=jax.ShapeDtypeStruct((2 * _NP, _HF), jnp.float32),
        mesh=_sc_mesh(),
        scratch_types=[
            pltpu.VMEM((_GRP, _CH), jnp.int32),
            pltpu.VMEM((_GRP, _CH), jnp.int32),
            pltpu.VMEM((_CH, _HF), jnp.float32),
            pltpu.VMEM((_CH, _HF), jnp.float32),
            pltpu.VMEM((_ZR, _HF), jnp.float32),
            pltpu.VMEM_SHARED((_NP, _HF), jnp.float32),
            pltpu.SemaphoreType.DMA,
            pltpu.SemaphoreType.DMA,
        ],
    )
    def k(g_hbm, srcs_hbm, dsts_hbm, z_hbm, out_hbm,
          sidx, didx, rows_a, rows_b, zbuf, acc, sem_a, sem_b):
        c = lax.axis_index("c")
        s = lax.axis_index("s")
        pltpu.sync_copy(z_hbm, zbuf)
        for t in range(_NZ):
            pltpu.sync_copy(zbuf, acc.at[pl.ds(s * _RPS + t * _ZR, _ZR)])
        plsc.subcore_barrier()

        def group(gi, carry):
            pltpu.sync_copy(srcs_hbm.at[c, s, gi], sidx)
            pltpu.sync_copy(dsts_hbm.at[s, gi], didx)

            def body(j, carry2):
                pltpu.async_copy(g_hbm.at[sidx.at[j]], rows_a, sem_a).wait()
                pltpu.sync_copy(rows_a, acc.at[didx.at[j]], add=True)
                return carry2

            lax.fori_loop(0, _GRP, body, 0)
            return carry

        lax.fori_loop(0, _NGRP, group, 0)
        plsc.subcore_barrier()
        for t in range(_NZ):
            pltpu.sync_copy(acc.at[pl.ds(s * _RPS + t * _ZR, _ZR)], zbuf)
            pltpu.sync_copy(
                zbuf, out_hbm.at[pl.ds(c * _NP + s * _RPS + t * _ZR, _ZR)])

    return k(g2, srcs2, dsts_s, z128)


def _ln(x, s, b):
    mu = jnp.mean(x, axis=-1, keepdims=True)
    var = jnp.mean((x - mu) ** 2, axis=-1, keepdims=True)
    return (x - mu) * lax.rsqrt(var + 1e-5) * s + b


def _split_halves(g, out_ref):
    out_ref[0] = g[:, :_HF]
    out_ref[1] = g[:, _HF:]


def _tc1_body(x_ref, w_in_ref, b_in_ref, w1_ref, deg2_ref,
              h0_ref, dinv_ref, g1s_ref):
    deg = deg2_ref[0, :, 0:1] + deg2_ref[1, :, 0:1] + 1.0
    dinv = lax.rsqrt(deg)
    h0 = jnp.dot(x_ref[...], w_in_ref[...],
                 preferred_element_type=jnp.float32) + b_in_ref[...]
    h0_ref[...] = h0
    dinv_ref[...] = dinv
    g1 = jnp.dot(h0, w1_ref[...], preferred_element_type=jnp.float32) * dinv
    _split_halves(g1, g1s_ref)


def _tc_mid_body(h_prev_ref, dinv_ref, s_ref, gs_ref, b_ref, ln_s_ref,
                 ln_b_ref, w_next_ref, h_ref, gn_ref):
    dinv = dinv_ref[...]
    s_full = jnp.concatenate([s_ref[0], s_ref[1]], axis=-1)
    g_full = jnp.concatenate([gs_ref[0], gs_ref[1]], axis=-1)
    out = dinv * (s_full + g_full) + b_ref[...]
    h = jax.nn.relu(_ln(out, ln_s_ref[...], ln_b_ref[...])) + h_prev_ref[...]
    h_ref[...] = h
    gn = jnp.dot(h, w_next_ref[...], preferred_element_type=jnp.float32) * dinv
    _split_halves(gn, gn_ref)


def _tc3a_body(h_prev_ref, dinv_ref, s_ref, gs_ref, b_ref, ln_s_ref, ln_b_ref,
               watt1_ref, batt1_ref, watt2_ref, batt2_ref, h_ref, scores_ref):
    dinv = dinv_ref[...]
    s_full = jnp.concatenate([s_ref[0], s_ref[1]], axis=-1)
    g_full = jnp.concatenate([gs_ref[0], gs_ref[1]], axis=-1)
    out = dinv * (s_full + g_full) + b_ref[...]
    h = jax.nn.relu(_ln(out, ln_s_ref[...], ln_b_ref[...])) + h_prev_ref[...]
    h_ref[...] = h

    batt1 = batt1_ref[...]
    batt2 = batt2_ref[...]
    cols = []
    for hd in range(4):
        inner = jnp.tanh(
            jnp.dot(h, watt1_ref[hd], preferred_element_type=jnp.float32)
            + batt1[hd:hd + 1, :])
        sc = jnp.dot(inner, watt2_ref[hd],
                     preferred_element_type=jnp.float32) + batt2[hd:hd + 1, :]
        cols.append(sc)
    scores_ref[...] = jnp.concatenate(cols, axis=-1)


def _tc3b_body(h_ref, scores_ref, wc1_ref, bc1_ref, wc2_ref, bc2_ref,
               probs_ref, attw_ref):
    h = h_ref[...]
    scores = scores_ref[...]
    m = jnp.max(scores, axis=0, keepdims=True)
    e = jnp.exp(scores - m)
    a = e / jnp.sum(e, axis=0, keepdims=True)
    attw_ref[...] = a

    z_agg = jnp.zeros((1, 256), jnp.float32)
    for hd in range(4):
        z_agg = z_agg + jnp.sum(a[:, hd:hd + 1] * h, axis=0, keepdims=True)
    z_agg = z_agg * 0.25
    hid = jax.nn.relu(
        jnp.dot(z_agg, wc1_ref[...], preferred_element_type=jnp.float32)
        + bc1_ref[...])
    logits = jnp.dot(hid, wc2_ref[...],
                     preferred_element_type=jnp.float32) + bc2_ref[...]
    lm = jnp.max(logits, axis=-1, keepdims=True)
    le = jnp.exp(logits - lm)
    probs_ref[...] = le / jnp.sum(le, axis=-1, keepdims=True)


_BLK = 2000


def _row_spec(shape):
    if shape is None:
        return pl.BlockSpec(None, lambda i: None)
    if len(shape) == 2 and shape[0] in (_N, _NP):
        return pl.BlockSpec((_BLK, shape[1]), lambda i: (i, 0))
    if len(shape) == 3 and shape[1] in (_N, _NP):
        return pl.BlockSpec((shape[0], _BLK, shape[2]), lambda i: (0, i, 0))
    return pl.BlockSpec(shape, lambda i: tuple(0 for _ in shape))


def _tc_row_call(body, out_shapes, *args):
    return pl.pallas_call(
        body,
        grid=(_N // _BLK,),
        in_specs=[_row_spec(a.shape) for a in args],
        out_specs=[_row_spec(s) for s in out_shapes],
        out_shape=[jax.ShapeDtypeStruct(s, jnp.float32) for s in out_shapes],
    )(*args)


def _tc_call(body, out_shapes, *args):
    return pl.pallas_call(
        body,
        out_shape=[jax.ShapeDtypeStruct(s, jnp.float32) for s in out_shapes],
    )(*args)


def kernel(x, edge_index, W_in, b_in, W1, b1, ln1_s, ln1_b, W2, b2,
           ln2_s, ln2_b, Watt1, batt1, Watt2, batt2, Wc1, bc1, Wc2, bc2):
    src = edge_index[0]
    dst = edge_index[1]
    srcs2 = jnp.stack([src, src + _N]).reshape(_NC, _NS, _NGRP, _GRP, _CH)
    dsts_s = dst.reshape(_NS, _NGRP, _GRP, _CH)
    dsts_d = dst.reshape(_NC * _NS, _DPW, _CH)
    z128 = jnp.zeros((_ZR, _HF), jnp.float32)
    ones128 = jnp.ones((_CH, _HF), jnp.float32)

    deg2 = _sc_degree(dsts_d, z128, ones128).reshape(_NC, _NP, _HF)

    b_in2 = b_in.reshape(1, -1)
    b1_2, b2_2 = b1.reshape(1, -1), b2.reshape(1, -1)
    ln1_s2, ln1_b2 = ln1_s.reshape(1, -1), ln1_b.reshape(1, -1)
    ln2_s2, ln2_b2 = ln2_s.reshape(1, -1), ln2_b.reshape(1, -1)
    bc1_2, bc2_2 = bc1.reshape(1, -1), bc2.reshape(1, -1)
    batt2_2 = batt2.reshape(4, 1)

    h0, dinv, g1s = _tc_row_call(
        _tc1_body, [(_N, 256), (_N, 1), (_NC, _N, _HF)],
        x, W_in, b_in2, W1, deg2)

    s1 = _sc_scatter(g1s.reshape(2 * _N, _HF), srcs2, dsts_s,
                     z128).reshape(_NC, _NP, _HF)

    h1, g2s = _tc_row_call(
        _tc_mid_body, [(_N, 256), (_NC, _N, _HF)],
        h0, dinv, s1, g1s, b1_2, ln1_s2, ln1_b2, W2)

    s2 = _sc_scatter(g2s.reshape(2 * _N, _HF), srcs2, dsts_s,
                     z128).reshape(_NC, _NP, _HF)

    h2, scores = _tc_row_call(
        _tc3a_body, [(_N, 256), (_N, 4)],
        h1, dinv, s2, g2s, b2_2, ln2_s2, ln2_b2,
        Watt1, batt1, Watt2, batt2_2)

    probs, att_w = _tc_call(
        _tc3b_body, [(1, 7), (_N, 4)],
        h2, scores, Wc1, bc1_2, Wc2, bc2_2)

    return probs.reshape(7), att_w


# speedup vs baseline: 17.7816x; 1.5321x over previous
import functools

import jax
import jax.numpy as jnp
from jax import lax
from jax.experimental import pallas as pl
from jax.experimental.pallas import tpu as pltpu
from jax.experimental.pallas import tpu_sc as plsc

_N = 10000
_NP = 10240
_E = 320000
_HF = 128
_NC, _NS = 2, 16
_CH = 80
_NCHUNK = _E // _CH
_CPS = _NCHUNK // _NS
_GRP = 50
_NGRP = _CPS // _GRP
_DPW = _NCHUNK // (_NC * _NS)
_RPS = _NP // _NS
_ZR = 64
_NZ = _RPS // _ZR


def _sc_mesh():
    return plsc.VectorSubcoreMesh(
        core_axis_name="c", subcore_axis_name="s",
        num_cores=_NC, num_subcores=_NS)


def _sc_degree(dsts_d, z128, ones128):
    @functools.partial(
        pl.kernel,
        out_type=jax.ShapeDtypeStruct((2 * _NP, _HF), jnp.float32),
        mesh=_sc_mesh(),
        scratch_types=[
            pltpu.VMEM((_DPW, _CH), jnp.int32),
            pltpu.VMEM((_CH, _HF), jnp.float32),
            pltpu.VMEM((_ZR, _HF), jnp.float32),
            pltpu.VMEM_SHARED((_NP, _HF), jnp.float32),
        ],
    )
    def k(dsts_hbm, z_hbm, ones_hbm, out_hbm, didx, ones_v, zbuf, acc):
        c = lax.axis_index("c")
        s = lax.axis_index("s")
        wid = c * _NS + s
        pltpu.sync_copy(dsts_hbm.at[wid], didx)
        pltpu.sync_copy(ones_hbm, ones_v)
        pltpu.sync_copy(z_hbm, zbuf)
        for t in range(_NZ):
            pltpu.sync_copy(zbuf, acc.at[pl.ds(s * _RPS + t * _ZR, _ZR)])
        plsc.subcore_barrier()

        def body(j, carry):
            pltpu.sync_copy(ones_v, acc.at[didx.at[j]], add=True)
            return carry

        lax.fori_loop(0, _DPW, body, 0)
        plsc.subcore_barrier()
        for t in range(_NZ):
            pltpu.sync_copy(acc.at[pl.ds(s * _RPS + t * _ZR, _ZR)], zbuf)
            pltpu.sync_copy(
                zbuf, out_hbm.at[pl.ds(c * _NP + s * _RPS + t * _ZR, _ZR)])

    return k(dsts_d, z128, ones128)


def _sc_scatter(g2, srcs2, dsts_s, z128):
    @functools.partial(
        pl.kernel,
        out_type=jax.ShapeDtypeStruct((2 * _NP, _HF), jnp.float32),
        mesh=_sc_mesh(),
        scratch_types=[
            pltpu.VMEM((_GRP, _CH), jnp.int32),
            pltpu.VMEM((_GRP, _CH), jnp.int32),
            pltpu.VMEM((_CH, _HF), jnp.float32),
            pltpu.VMEM((_CH, _HF), jnp.float32),
            pltpu.VMEM((_ZR, _HF), jnp.float32),
            pltpu.VMEM_SHARED((_NP, _HF), jnp.float32),
            pltpu.SemaphoreType.DMA,
            pltpu.SemaphoreType.DMA,
        ],
    )
    def k(g_hbm, srcs_hbm, dsts_hbm, z_hbm, out_hbm,
          sidx, didx, rows_a, rows_b, zbuf, acc, sem_a, sem_b):
        c = lax.axis_index("c")
        s = lax.axis_index("s")
        pltpu.sync_copy(z_hbm, zbuf)
        for t in range(_NZ):
            pltpu.sync_copy(zbuf, acc.at[pl.ds(s * _RPS + t * _ZR, _ZR)])
        plsc.subcore_barrier()

        def group(gi, carry):
            pltpu.sync_copy(srcs_hbm.at[c, s, gi], sidx)
            pltpu.sync_copy(dsts_hbm.at[s, gi], didx)
            pltpu.async_copy(g_hbm.at[sidx.at[0]], rows_a, sem_a)

            def body(j, carry2):
                pltpu.async_copy(g_hbm.at[sidx.at[2 * j + 1]], rows_b, sem_b)
                pltpu.make_async_copy(
                    g_hbm.at[sidx.at[0]], rows_a, sem_a).wait()
                pltpu.sync_copy(rows_a, acc.at[didx.at[2 * j]], add=True)

                @pl.when(j + 1 < _GRP // 2)
                def _():
                    pltpu.async_copy(
                        g_hbm.at[sidx.at[2 * j + 2]], rows_a, sem_a)

                pltpu.make_async_copy(
                    g_hbm.at[sidx.at[0]], rows_b, sem_b).wait()
                pltpu.sync_copy(rows_b, acc.at[didx.at[2 * j + 1]], add=True)
                return carry2

            lax.fori_loop(0, _GRP // 2, body, 0)
            return carry

        lax.fori_loop(0, _NGRP, group, 0)
        plsc.subcore_barrier()
        for t in range(_NZ):
            pltpu.sync_copy(acc.at[pl.ds(s * _RPS + t * _ZR, _ZR)], zbuf)
            pltpu.sync_copy(
                zbuf, out_hbm.at[pl.ds(c * _NP + s * _RPS + t * _ZR, _ZR)])

    return k(g2, srcs2, dsts_s, z128)


def _ln(x, s, b):
    mu = jnp.mean(x, axis=-1, keepdims=True)
    var = jnp.mean((x - mu) ** 2, axis=-1, keepdims=True)
    return (x - mu) * lax.rsqrt(var + 1e-5) * s + b


def _split_halves(g, out_ref):
    out_ref[0] = g[:, :_HF]
    out_ref[1] = g[:, _HF:]


def _tc1_body(x_ref, w_in_ref, b_in_ref, w1_ref, deg2_ref,
              h0_ref, dinv_ref, g1s_ref):
    deg = deg2_ref[0, :, 0:1] + deg2_ref[1, :, 0:1] + 1.0
    dinv = lax.rsqrt(deg)
    h0 = jnp.dot(x_ref[...], w_in_ref[...],
                 preferred_element_type=jnp.float32) + b_in_ref[...]
    h0_ref[...] = h0
    dinv_ref[...] = dinv
    g1 = jnp.dot(h0, w1_ref[...], preferred_element_type=jnp.float32) * dinv
    _split_halves(g1, g1s_ref)


def _tc_mid_body(h_prev_ref, dinv_ref, s_ref, gs_ref, b_ref, ln_s_ref,
                 ln_b_ref, w_next_ref, h_ref, gn_ref):
    dinv = dinv_ref[...]
    s_full = jnp.concatenate([s_ref[0], s_ref[1]], axis=-1)
    g_full = jnp.concatenate([gs_ref[0], gs_ref[1]], axis=-1)
    out = dinv * (s_full + g_full) + b_ref[...]
    h = jax.nn.relu(_ln(out, ln_s_ref[...], ln_b_ref[...])) + h_prev_ref[...]
    h_ref[...] = h
    gn = jnp.dot(h, w_next_ref[...], preferred_element_type=jnp.float32) * dinv
    _split_halves(gn, gn_ref)


def _tc3a_body(h_prev_ref, dinv_ref, s_ref, gs_ref, b_ref, ln_s_ref, ln_b_ref,
               watt1_ref, batt1_ref, watt2_ref, batt2_ref, h_ref, scores_ref):
    dinv = dinv_ref[...]
    s_full = jnp.concatenate([s_ref[0], s_ref[1]], axis=-1)
    g_full = jnp.concatenate([gs_ref[0], gs_ref[1]], axis=-1)
    out = dinv * (s_full + g_full) + b_ref[...]
    h = jax.nn.relu(_ln(out, ln_s_ref[...], ln_b_ref[...])) + h_prev_ref[...]
    h_ref[...] = h

    batt1 = batt1_ref[...]
    batt2 = batt2_ref[...]
    cols = []
    for hd in range(4):
        inner = jnp.tanh(
            jnp.dot(h, watt1_ref[hd], preferred_element_type=jnp.float32)
            + batt1[hd:hd + 1, :])
        sc = jnp.dot(inner, watt2_ref[hd],
                     preferred_element_type=jnp.float32) + batt2[hd:hd + 1, :]
        cols.append(sc)
    scores_ref[...] = jnp.concatenate(cols, axis=-1)


def _tc3b_body(h_ref, scores_ref, wc1_ref, bc1_ref, wc2_ref, bc2_ref,
               probs_ref, attw_ref):
    h = h_ref[...]
    scores = scores_ref[...]
    m = jnp.max(scores, axis=0, keepdims=True)
    e = jnp.exp(scores - m)
    a = e / jnp.sum(e, axis=0, keepdims=True)
    attw_ref[...] = a

    z_agg = jnp.zeros((1, 256), jnp.float32)
    for hd in range(4):
        z_agg = z_agg + jnp.sum(a[:, hd:hd + 1] * h, axis=0, keepdims=True)
    z_agg = z_agg * 0.25
    hid = jax.nn.relu(
        jnp.dot(z_agg, wc1_ref[...], preferred_element_type=jnp.float32)
        + bc1_ref[...])
    logits = jnp.dot(hid, wc2_ref[...],
                     preferred_element_type=jnp.float32) + bc2_ref[...]
    lm = jnp.max(logits, axis=-1, keepdims=True)
    le = jnp.exp(logits - lm)
    probs_ref[...] = le / jnp.sum(le, axis=-1, keepdims=True)


_BLK = 2000


def _row_spec(shape):
    if shape is None:
        return pl.BlockSpec(None, lambda i: None)
    if len(shape) == 2 and shape[0] in (_N, _NP):
        return pl.BlockSpec((_BLK, shape[1]), lambda i: (i, 0))
    if len(shape) == 3 and shape[1] in (_N, _NP):
        return pl.BlockSpec((shape[0], _BLK, shape[2]), lambda i: (0, i, 0))
    return pl.BlockSpec(shape, lambda i: tuple(0 for _ in shape))


def _tc_row_call(body, out_shapes, *args):
    return pl.pallas_call(
        body,
        grid=(_N // _BLK,),
        in_specs=[_row_spec(a.shape) for a in args],
        out_specs=[_row_spec(s) for s in out_shapes],
        out_shape=[jax.ShapeDtypeStruct(s, jnp.float32) for s in out_shapes],
    )(*args)


def _tc_call(body, out_shapes, *args):
    return pl.pallas_call(
        body,
        out_shape=[jax.ShapeDtypeStruct(s, jnp.float32) for s in out_shapes],
    )(*args)


def kernel(x, edge_index, W_in, b_in, W1, b1, ln1_s, ln1_b, W2, b2,
           ln2_s, ln2_b, Watt1, batt1, Watt2, batt2, Wc1, bc1, Wc2, bc2):
    src = edge_index[0]
    dst = edge_index[1]
    srcs2 = jnp.stack([src, src + _N]).reshape(_NC, _NS, _NGRP, _GRP, _CH)
    dsts_s = dst.reshape(_NS, _NGRP, _GRP, _CH)
    dsts_d = dst.reshape(_NC * _NS, _DPW, _CH)
    z128 = jnp.zeros((_ZR, _HF), jnp.float32)
    ones128 = jnp.ones((_CH, _HF), jnp.float32)

    deg2 = _sc_degree(dsts_d, z128, ones128).reshape(_NC, _NP, _HF)

    b_in2 = b_in.reshape(1, -1)
    b1_2, b2_2 = b1.reshape(1, -1), b2.reshape(1, -1)
    ln1_s2, ln1_b2 = ln1_s.reshape(1, -1), ln1_b.reshape(1, -1)
    ln2_s2, ln2_b2 = ln2_s.reshape(1, -1), ln2_b.reshape(1, -1)
    bc1_2, bc2_2 = bc1.reshape(1, -1), bc2.reshape(1, -1)
    batt2_2 = batt2.reshape(4, 1)

    h0, dinv, g1s = _tc_row_call(
        _tc1_body, [(_N, 256), (_N, 1), (_NC, _N, _HF)],
        x, W_in, b_in2, W1, deg2)

    s1 = _sc_scatter(g1s.reshape(2 * _N, _HF), srcs2, dsts_s,
                     z128).reshape(_NC, _NP, _HF)

    h1, g2s = _tc_row_call(
        _tc_mid_body, [(_N, 256), (_NC, _N, _HF)],
        h0, dinv, s1, g1s, b1_2, ln1_s2, ln1_b2, W2)

    s2 = _sc_scatter(g2s.reshape(2 * _N, _HF), srcs2, dsts_s,
                     z128).reshape(_NC, _NP, _HF)

    h2, scores = _tc_row_call(
        _tc3a_body, [(_N, 256), (_N, 4)],
        h1, dinv, s2, g2s, b2_2, ln2_s2, ln2_b2,
        Watt1, batt1, Watt2, batt2_2)

    probs, att_w = _tc_call(
        _tc3b_body, [(1, 7), (_N, 4)],
        h2, scores, Wc1, bc1_2, Wc2, bc2_2)

    return probs.reshape(7), att_w
